# Initial kernel scaffold; baseline (speedup 1.0000x reference)
import jax
import jax.numpy as jnp
from jax.experimental import pallas as pl


def kernel(src_neighbor_ids, dst_neighbor_ids, W1, b1, W2, b2):
    raise NotImplementedError("write your pallas kernel here")


# speedup vs baseline: 5.3118x; 5.3118x over previous
import functools

import jax
import jax.numpy as jnp
from jax import lax
from jax.experimental import pallas as pl
from jax.experimental.pallas import tpu as pltpu
from jax.experimental.pallas import tpu_sc as plsc

B = 4096
L = 200
V = 1000
D = 64

NUM_WORKERS = 32
ROWS_PER_W = B // NUM_WORKERS
CR = 32
NCHUNK = ROWS_PER_W // CR
HSIZE = 1008
LANES = 16
NFULL = L // LANES
TAIL = L - NFULL * LANES
ROW_W = CR * L


def _sc_counts_kernel(src_hbm, dst_hbm, o_ss, o_sd, o_ds, o_dd,
                      sv, dv, hs, hd, v_ss, v_sd, v_ds, v_dd):
    cid = lax.axis_index("c")
    sid = lax.axis_index("s")
    wid = sid * 2 + cid

    zf = jnp.zeros((LANES,), jnp.float32)
    zi = jnp.zeros((LANES,), jnp.int32)
    tail_mask = lax.iota(jnp.int32, LANES) < TAIL

    for i in range(HSIZE // LANES):
        hs[pl.ds(i * LANES, LANES)] = zf
        hd[pl.ds(i * LANES, LANES)] = zf
    sv[pl.ds(ROW_W, LANES)] = zi
    dv[pl.ds(ROW_W, LANES)] = zi

    for chunk in range(NCHUNK):
        gbase = (wid * ROWS_PER_W + chunk * CR) * L
        pltpu.sync_copy(src_hbm.at[pl.ds(gbase, ROW_W)], sv.at[pl.ds(0, ROW_W)])
        pltpu.sync_copy(dst_hbm.at[pl.ds(gbase, ROW_W)], dv.at[pl.ds(0, ROW_W)])

        def row_body(r, carry):
            rb = r * L

            for ids, h in ((sv, hs), (dv, hd)):
                for j in range(NFULL):
                    idx = ids[pl.ds(rb + j * LANES, LANES)]
                    cnt, last = plsc.scan_count(idx)
                    plsc.addupdate_scatter(h, [idx], cnt.astype(jnp.float32),
                                           mask=last)
                idx = ids[pl.ds(rb + NFULL * LANES, LANES)]
                cnt, last = plsc.scan_count(idx, tail_mask)
                plsc.addupdate_scatter(h, [idx], cnt.astype(jnp.float32),
                                       mask=last)

            for off in tuple(range(0, NFULL * LANES, LANES)) + (L - LANES,):
                si = sv[pl.ds(rb + off, LANES)]
                di = dv[pl.ds(rb + off, LANES)]
                c_ss = plsc.load_gather(hs, [si])
                c_sd = plsc.load_gather(hd, [si])
                c_ds = plsc.load_gather(hs, [di])
                c_dd = plsc.load_gather(hd, [di])
                sm = si != 0
                dm = di != 0
                v_ss[pl.ds(rb + off, LANES)] = jnp.where(sm, c_ss, 0.0)
                v_sd[pl.ds(rb + off, LANES)] = jnp.where(sm, c_sd, 0.0)
                v_ds[pl.ds(rb + off, LANES)] = jnp.where(dm, c_ds, 0.0)
                v_dd[pl.ds(rb + off, LANES)] = jnp.where(dm, c_dd, 0.0)

            for ids, h in ((sv, hs), (dv, hd)):
                for j in range(NFULL + 1):
                    idx = ids[pl.ds(rb + j * LANES, LANES)]
                    plsc.store_scatter(h, [idx], zf)
            return carry

        lax.fori_loop(0, CR, row_body, 0)

        pltpu.sync_copy(v_ss, o_ss.at[pl.ds(gbase, ROW_W)])
        pltpu.sync_copy(v_sd, o_sd.at[pl.ds(gbase, ROW_W)])
        pltpu.sync_copy(v_ds, o_ds.at[pl.ds(gbase, ROW_W)])
        pltpu.sync_copy(v_dd, o_dd.at[pl.ds(gbase, ROW_W)])


def _sc_counts(src_flat, dst_flat):
    mesh = plsc.VectorSubcoreMesh(core_axis_name="c", subcore_axis_name="s")
    n = B * L
    out = jax.ShapeDtypeStruct((n,), jnp.float32)
    f = pl.kernel(
        _sc_counts_kernel,
        out_type=[out, out, out, out],
        mesh=mesh,
        compiler_params=pltpu.CompilerParams(needs_layout_passes=False),
        scratch_types=[
            pltpu.VMEM((ROW_W + LANES,), jnp.int32),
            pltpu.VMEM((ROW_W + LANES,), jnp.int32),
            pltpu.VMEM((HSIZE,), jnp.float32),
            pltpu.VMEM((HSIZE,), jnp.float32),
            pltpu.VMEM((ROW_W,), jnp.float32),
            pltpu.VMEM((ROW_W,), jnp.float32),
            pltpu.VMEM((ROW_W,), jnp.float32),
            pltpu.VMEM((ROW_W,), jnp.float32),
        ],
    )
    return f(src_flat, dst_flat)


TOK_LANE = 1024
SUBROWS = 8
TOK_BLK = SUBROWS * TOK_LANE
NT = B * L
GRID = NT // TOK_BLK


def _mlp_kernel(css, csd, cds, cdd, w1b, w2, b2, src_out, dst_out):
    ones = jnp.ones((1, TOK_LANE), jnp.float32)
    w1b_v = w1b[...]
    w2_v = w2[...]
    b2_v = b2[...]
    for s in range(SUBROWS):
        for (c1, c2, out) in ((css, csd, src_out), (cds, cdd, dst_out)):
            x = jnp.concatenate(
                [c1[pl.ds(s, 1), :], c2[pl.ds(s, 1), :], ones], axis=0)
            h = lax.dot_general(w1b_v, x, (((0,), (0,)), ((), ())),
                                preferred_element_type=jnp.float32)
            h = jnp.maximum(h, 0.0)
            y = lax.dot_general(h, w2_v, (((0,), (0,)), ((), ())),
                                preferred_element_type=jnp.float32)
            out[pl.ds(s * TOK_LANE, TOK_LANE), :] = y + b2_v


def _mlp(css, csd, cds, cdd, W1, b1, W2, b2):
    w1b = jnp.concatenate([W1, b1[None, :]], axis=0)
    b2r = b2[None, :]
    c2d = lambda c: c.reshape(NT // TOK_LANE, TOK_LANE)
    cspec = pl.BlockSpec((SUBROWS, TOK_LANE), lambda i: (i, 0))
    wspec = lambda shape: pl.BlockSpec(shape, lambda i: (0, 0))
    ospec = pl.BlockSpec((TOK_BLK, D), lambda i: (i, 0))
    out = jax.ShapeDtypeStruct((NT, D), jnp.float32)
    return pl.pallas_call(
        _mlp_kernel,
        grid=(GRID,),
        in_specs=[cspec, cspec, cspec, cspec,
                  wspec((3, D)), wspec((D, D)), wspec((1, D))],
        out_specs=[ospec, ospec],
        out_shape=[out, out],
        compiler_params=pltpu.CompilerParams(
            dimension_semantics=("arbitrary",)),
    )(c2d(css), c2d(csd), c2d(cds), c2d(cdd), w1b, W2, b2r)


def kernel(src_neighbor_ids, dst_neighbor_ids, W1, b1, W2, b2):
    src_flat = src_neighbor_ids.astype(jnp.int32).reshape(-1)
    dst_flat = dst_neighbor_ids.astype(jnp.int32).reshape(-1)
    c_ss, c_sd, c_ds, c_dd = _sc_counts(src_flat, dst_flat)
    src_feat, dst_feat = _mlp(c_ss, c_sd, c_ds, c_dd, W1, b1, W2, b2)
    return (src_feat.reshape(B, L, D), dst_feat.reshape(B, L, D))
